# Initial kernel scaffold; baseline (speedup 1.0000x reference)
import jax
import jax.numpy as jnp
from jax.experimental import pallas as pl


def kernel(shower, embeddings, edge_index):
    raise NotImplementedError("write your pallas kernel here")


# speedup vs baseline: 1.7808x; 1.7808x over previous
import functools

import jax
import jax.numpy as jnp
from jax import lax
from jax.experimental import pallas as pl
from jax.experimental.pallas import tpu as pltpu
from jax.experimental.pallas import tpu_sc as plsc

_R = 2.0
_T = 1.0

_NC = 2
_NS = 16
_NW = _NC * _NS
_L = 16
_B = 64


def _sc_body(n_chunks, emb_hbm, src_hbm, dst_hbm, out_hbm,
             idxs_v, idxd_v, srcbuf, dstbuf, sums_v, outw_v, sem1, sem2):
    ew = n_chunks * _B
    d = srcbuf.shape[1]
    nj = d // _L
    wid = lax.axis_index("s") * _NC + lax.axis_index("c")
    base = wid * ew

    pltpu.sync_copy(src_hbm.at[pl.ds(base, ew)], idxs_v)
    pltpu.sync_copy(dst_hbm.at[pl.ds(base, ew)], idxd_v)

    lane15 = lax.iota(jnp.int32, _L) * _L + (_L - 1)

    def chunk_body(c, carry):
        cs = c * _B
        cp1 = pltpu.async_copy(emb_hbm.at[idxs_v.at[pl.ds(cs, _B)]], srcbuf, sem1)
        cp2 = pltpu.async_copy(emb_hbm.at[idxd_v.at[pl.ds(cs, _B)]], dstbuf, sem2)
        cp1.wait()
        cp2.wait()

        def edge_body(e, carry2):
            acc = jnp.zeros((_L,), jnp.float32)
            for j in range(nj):
                a = srcbuf[e, pl.ds(j * _L, _L)]
                b = dstbuf[e, pl.ds(j * _L, _L)]
                diff = a - b
                acc = acc + diff * diff
            sums_v[pl.ds(e * _L, _L)] = plsc.cumsum(acc)
            return carry2

        lax.fori_loop(0, _B, edge_body, 0, unroll=2)

        def group_body(g, carry2):
            idx = g * (_L * _L) + lane15
            s = plsc.load_gather(sums_v, [idx])
            probs = 1.0 / (jnp.exp((s - _R) * (1.0 / _T)) + 1.0)
            outw_v[pl.ds(cs + g * _L, _L)] = probs
            return carry2

        lax.fori_loop(0, _B // _L, group_body, 0)
        return carry

    lax.fori_loop(0, n_chunks, chunk_body, 0)
    pltpu.sync_copy(outw_v, out_hbm.at[pl.ds(base, ew)])


@jax.jit
def kernel(shower, embeddings, edge_index):
    del shower
    e_total = edge_index.shape[1]
    d = embeddings.shape[1]
    ew = (-(-e_total // _NW) + _B - 1) // _B * _B
    n_chunks = ew // _B
    e_pad = ew * _NW

    src = edge_index[0]
    dst = edge_index[1]
    pad = e_pad - e_total
    if pad:
        zpad = jnp.zeros((pad,), jnp.int32)
        src = jnp.concatenate([src, zpad])
        dst = jnp.concatenate([dst, zpad])

    mesh = plsc.VectorSubcoreMesh(core_axis_name="c", subcore_axis_name="s")
    fn = pl.kernel(
        functools.partial(_sc_body, n_chunks),
        out_type=jax.ShapeDtypeStruct((e_pad,), jnp.float32),
        mesh=mesh,
        compiler_params=pltpu.CompilerParams(needs_layout_passes=False),
        scratch_types=[
            pltpu.VMEM((ew,), jnp.int32),
            pltpu.VMEM((ew,), jnp.int32),
            pltpu.VMEM((_B, d), jnp.float32),
            pltpu.VMEM((_B, d), jnp.float32),
            pltpu.VMEM((_B * _L,), jnp.float32),
            pltpu.VMEM((ew,), jnp.float32),
            pltpu.SemaphoreType.DMA,
            pltpu.SemaphoreType.DMA,
        ],
    )
    out = fn(embeddings, src, dst)
    return out[:e_total]
